# baseline (speedup 1.0000x reference)
import functools

import jax
import jax.numpy as jnp
from jax import lax
from jax.experimental import pallas as pl
from jax.experimental.pallas import tpu as pltpu
from jax.experimental.pallas import tpu_sc as plsc

VOCAB = 100000
D = 64
BATCH = 1024
STATES = 20
ATOMS = 20
PER_ROW = ATOMS * 3
ROW_PAD = 64
NC = 2
NS = 16
NW = NC * NS
B_PER_W = BATCH // NW
ACT_PER_W = B_PER_W * STATES
R_BLK = 2
IDX_BLK = R_BLK * ROW_PAD
OBS_BLK = B_PER_W // R_BLK
N_BLK = OBS_BLK + ACT_PER_W // R_BLK
OBS_W_IDX = B_PER_W * ROW_PAD
ACT_W_IDX = ACT_PER_W * ROW_PAD
NBUF = 4
LANES = 16
NCH = D // LANES
J_GRP = 15
SCALE = 1.0 / ATOMS


@functools.partial(
    pl.kernel,
    mesh=plsc.VectorSubcoreMesh(core_axis_name="c", subcore_axis_name="s"),
    out_type=(jax.ShapeDtypeStruct((BATCH, D), jnp.float32),
              jax.ShapeDtypeStruct((BATCH * STATES, D), jnp.float32)),
    compiler_params=pltpu.CompilerParams(use_tc_tiling_on_sc=False),
    scratch_types=[
        pltpu.VMEM((OBS_W_IDX + ACT_W_IDX,), jnp.int32),
        [pltpu.VMEM((IDX_BLK, D), jnp.float32) for _ in range(NBUF)],
        [pltpu.VMEM((R_BLK, D), jnp.float32) for _ in range(NBUF)],
        [pltpu.SemaphoreType.DMA for _ in range(NBUF)],
        [pltpu.SemaphoreType.DMA for _ in range(NBUF)],
    ],
)
def _pooled_lookup(obs_idx_hbm, act_idx_hbm, table_hbm, obs_out_hbm,
                   act_out_hbm, idx_all, rows_bufs, out_bufs, semg, semo):
    wid = lax.axis_index("s") * NC + lax.axis_index("c")
    b0 = wid * B_PER_W

    pltpu.sync_copy(obs_idx_hbm.at[pl.ds(wid * OBS_W_IDX, OBS_W_IDX)],
                    idx_all.at[pl.ds(0, OBS_W_IDX)])
    pltpu.sync_copy(act_idx_hbm.at[pl.ds(wid * ACT_W_IDX, ACT_W_IDX)],
                    idx_all.at[pl.ds(OBS_W_IDX, ACT_W_IDX)])

    def gather(i, rows_b, sem_b):
        return pltpu.make_async_copy(
            table_hbm.at[idx_all.at[pl.ds(i * IDX_BLK, IDX_BLK)]],
            rows_b, sem_b)

    def start_out_store(i, out_b, sem_b):
        @pl.when(i < OBS_BLK)
        def _():
            pltpu.make_async_copy(
                out_b, obs_out_hbm.at[pl.ds(b0 + i * R_BLK, R_BLK)],
                sem_b).start()

        @pl.when(i >= OBS_BLK)
        def _():
            row = b0 * STATES + (i - OBS_BLK) * R_BLK
            pltpu.make_async_copy(
                out_b, act_out_hbm.at[pl.ds(row, R_BLK)], sem_b).start()

    def wait_out_store(out_b, sem_b):
        pltpu.make_async_copy(
            out_b, obs_out_hbm.at[pl.ds(0, R_BLK)], sem_b).wait()

    for b in range(NBUF):
        gather(b, rows_bufs[b], semg[b]).start()

    zeros = jnp.zeros((LANES,), jnp.float32)

    def body(p, carry):
        for b in range(NBUF):
            rows_b, out_b, semg_b, semo_b = (
                rows_bufs[b], out_bufs[b], semg[b], semo[b])
            i = NBUF * p + b
            gather(i, rows_b, semg_b).wait()

            def jbody(jj, accs):
                accs = list(accs)
                for u in range(J_GRP):
                    for r in range(R_BLK):
                        row = r * ROW_PAD + jj * J_GRP + u
                        for c in range(NCH):
                            accs[r * NCH + c] = accs[r * NCH + c] + (
                                rows_b[row, pl.ds(c * LANES, LANES)])
                return tuple(accs)

            accs = lax.fori_loop(0, PER_ROW // J_GRP, jbody,
                                 (zeros,) * (R_BLK * NCH))

            @pl.when(i + NBUF < N_BLK)
            def _():
                gather(i + NBUF, rows_b, semg_b).start()

            @pl.when(i >= NBUF)
            def _():
                wait_out_store(out_b, semo_b)

            for r in range(R_BLK):
                for c in range(NCH):
                    out_b[r, pl.ds(c * LANES, LANES)] = (
                        accs[r * NCH + c] * SCALE)
            start_out_store(i, out_b, semo_b)
        return carry

    lax.fori_loop(0, N_BLK // NBUF, body, 0)
    for b in range(NBUF):
        wait_out_store(out_bufs[b], semo[b])


def kernel(sub_index, derived_sub_indices, action_mask, table):
    obs2 = sub_index.reshape(BATCH, PER_ROW)
    act2 = derived_sub_indices.reshape(BATCH * STATES, PER_ROW)
    obs_pad = jnp.concatenate([obs2, obs2[:, :ROW_PAD - PER_ROW]], axis=1)
    act_pad = jnp.concatenate([act2, act2[:, :ROW_PAD - PER_ROW]], axis=1)
    obs, act = _pooled_lookup(obs_pad.reshape(-1), act_pad.reshape(-1), table)
    return (obs, act.reshape(BATCH, STATES, D), action_mask)
